# Initial kernel scaffold; baseline (speedup 1.0000x reference)
import functools

import jax
import jax.numpy as jnp
from jax import lax
from jax.experimental import pallas as pl
from jax.experimental.pallas import tpu as pltpu
from jax.experimental.pallas import tpu_sc as plsc

_BETA = 0.25
_ROW_BLOCK = 256


def _dist_argmin_body(nr, n_elems, z_ref, cbt_ref, idx_ref, loss_ref, esq_ref):
    i = pl.program_id(0)

    @pl.when(i == 0)
    def _init():
        esq_ref[...] = jnp.sum(cbt_ref[...] * cbt_ref[...], axis=0,
                               keepdims=True)
        loss_ref[...] = jnp.zeros_like(loss_ref)

    zb = z_ref[...]
    zsq = jnp.sum(zb * zb, axis=1, keepdims=True)
    scores = lax.dot_general(
        zb, cbt_ref[...], (((1,), (0,)), ((), ())),
        preferred_element_type=jnp.float32,
        precision=lax.Precision.HIGHEST)
    dist = jnp.maximum(zsq - 2.0 * scores + esq_ref[...], 0.0)
    k = dist.shape[1]
    mind = jnp.min(dist, axis=1, keepdims=True)
    iota = lax.broadcasted_iota(jnp.int32, dist.shape, 1)
    idx = jnp.min(jnp.where(dist == mind, iota, k), axis=1)
    idx_ref[...] = idx.reshape(idx_ref.shape)
    loss_ref[...] += jnp.sum(mind).reshape(1, 1)

    @pl.when(i == nr - 1)
    def _finish():
        loss_ref[...] = loss_ref[...] * ((1.0 + _BETA) / n_elems)


def _dist_argmin(z_flat, cbt):
    n, d = z_flat.shape
    k = cbt.shape[1]
    r = _ROW_BLOCK
    nr = n // r
    idx3, loss = pl.pallas_call(
        functools.partial(_dist_argmin_body, nr, n * d),
        grid=(nr,),
        in_specs=[
            pl.BlockSpec((r, d), lambda i: (i, 0)),
            pl.BlockSpec((d, k), lambda i: (0, 0)),
        ],
        out_specs=[
            pl.BlockSpec((1, 1, r), lambda i: (i, 0, 0)),
            pl.BlockSpec((1, 1), lambda i: (0, 0)),
        ],
        out_shape=[
            jax.ShapeDtypeStruct((nr, 1, r), jnp.int32),
            jax.ShapeDtypeStruct((1, 1), jnp.float32),
        ],
        scratch_shapes=[pltpu.VMEM((1, k), jnp.float32)],
        compiler_params=pltpu.CompilerParams(
            dimension_semantics=("arbitrary",)),
    )(z_flat, cbt)
    return idx3.reshape(n), loss[0, 0]


def _sc_gather(table, idx):
    n = idx.shape[0]
    d = table.shape[1]
    info = plsc.get_sparse_core_info()
    nw = info.num_cores * info.num_subcores
    b_per_w = n // nw
    mesh = plsc.VectorSubcoreMesh(core_axis_name="c", subcore_axis_name="s")

    @functools.partial(
        pl.kernel, mesh=mesh,
        out_type=jax.ShapeDtypeStruct((n, d), jnp.float32),
        scratch_types=[
            pltpu.VMEM((b_per_w,), jnp.int32),
            pltpu.VMEM((b_per_w, d), jnp.float32),
            pltpu.SemaphoreType.DMA,
        ],
    )
    def gather_kernel(table_hbm, idx_hbm, out_hbm, idx_v, rows_v, sem):
        wid = lax.axis_index("s") * info.num_cores + lax.axis_index("c")
        base = wid * b_per_w
        pltpu.sync_copy(idx_hbm.at[pl.ds(base, b_per_w)], idx_v)
        pltpu.async_copy(table_hbm.at[idx_v], rows_v, sem).wait()
        pltpu.sync_copy(rows_v, out_hbm.at[pl.ds(base, b_per_w)])

    return gather_kernel(table, idx)


def kernel(z, codebook):
    b, c, h, w = z.shape
    z_flat = jnp.transpose(z, (0, 2, 3, 1)).reshape(-1, c)
    cbt = codebook.astype(jnp.float32).T
    idx_flat, vq_loss = _dist_argmin(z_flat.astype(jnp.float32), cbt)
    z_q_flat = _sc_gather(codebook, idx_flat)
    z_q = jnp.transpose(z_q_flat.reshape(b, h, w, c), (0, 3, 1, 2))
    z_q_st = z + lax.stop_gradient(z_q - z)
    return (z_q_st, vq_loss, idx_flat.reshape(b, h, w))


# speedup vs baseline: 1.0043x; 1.0043x over previous
import functools

import jax
import jax.numpy as jnp
from jax import lax
from jax.experimental import pallas as pl
from jax.experimental.pallas import tpu as pltpu
from jax.experimental.pallas import tpu_sc as plsc

_BETA = 0.25
_ROW_BLOCK = 256


def _dist_argmin_body(nr, n_elems, z_ref, cbt_ref, idx_ref, loss_ref,
                      esq_ref, cbt16_ref):
    i = pl.program_id(0)

    @pl.when(i == 0)
    def _init():
        cbt = cbt_ref[...]
        esq_ref[...] = jnp.sum(cbt * cbt, axis=0, keepdims=True)
        cbt16_ref[...] = cbt.astype(jnp.bfloat16)
        loss_ref[...] = jnp.zeros_like(loss_ref)

    zb = z_ref[...]
    zsq = jnp.sum(zb * zb, axis=1, keepdims=True)
    scores = lax.dot_general(
        zb.astype(jnp.bfloat16), cbt16_ref[...], (((1,), (0,)), ((), ())),
        preferred_element_type=jnp.float32)
    dist = jnp.maximum(zsq - 2.0 * scores + esq_ref[...], 0.0)
    k = dist.shape[1]
    mind = jnp.min(dist, axis=1, keepdims=True)
    iota = lax.broadcasted_iota(jnp.int32, dist.shape, 1)
    idx = jnp.min(jnp.where(dist == mind, iota, k), axis=1)
    idx_ref[...] = idx.reshape(idx_ref.shape)
    loss_ref[...] += jnp.sum(mind).reshape(1, 1)

    @pl.when(i == nr - 1)
    def _finish():
        loss_ref[...] = loss_ref[...] * ((1.0 + _BETA) / n_elems)


def _dist_argmin(z_flat, cbt):
    n, d = z_flat.shape
    k = cbt.shape[1]
    r = _ROW_BLOCK
    nr = n // r
    idx3, loss = pl.pallas_call(
        functools.partial(_dist_argmin_body, nr, n * d),
        grid=(nr,),
        in_specs=[
            pl.BlockSpec((r, d), lambda i: (i, 0)),
            pl.BlockSpec((d, k), lambda i: (0, 0)),
        ],
        out_specs=[
            pl.BlockSpec((1, 1, r), lambda i: (i, 0, 0)),
            pl.BlockSpec((1, 1), lambda i: (0, 0)),
        ],
        out_shape=[
            jax.ShapeDtypeStruct((nr, 1, r), jnp.int32),
            jax.ShapeDtypeStruct((1, 1), jnp.float32),
        ],
        scratch_shapes=[pltpu.VMEM((1, k), jnp.float32),
                        pltpu.VMEM((d, k), jnp.bfloat16)],
        compiler_params=pltpu.CompilerParams(
            dimension_semantics=("arbitrary",)),
    )(z_flat, cbt)
    return idx3.reshape(n), loss[0, 0]


def _sc_gather(table, idx):
    n = idx.shape[0]
    d = table.shape[1]
    info = plsc.get_sparse_core_info()
    nw = info.num_cores * info.num_subcores
    b_per_w = n // nw
    mesh = plsc.VectorSubcoreMesh(core_axis_name="c", subcore_axis_name="s")

    @functools.partial(
        pl.kernel, mesh=mesh,
        out_type=jax.ShapeDtypeStruct((n, d), jnp.float32),
        scratch_types=[
            pltpu.VMEM((b_per_w,), jnp.int32),
            pltpu.VMEM((b_per_w, d), jnp.float32),
            pltpu.SemaphoreType.DMA,
        ],
    )
    def gather_kernel(table_hbm, idx_hbm, out_hbm, idx_v, rows_v, sem):
        wid = lax.axis_index("s") * info.num_cores + lax.axis_index("c")
        base = wid * b_per_w
        pltpu.sync_copy(idx_hbm.at[pl.ds(base, b_per_w)], idx_v)
        pltpu.async_copy(table_hbm.at[idx_v], rows_v, sem).wait()
        pltpu.sync_copy(rows_v, out_hbm.at[pl.ds(base, b_per_w)])

    return gather_kernel(table, idx)


def kernel(z, codebook):
    b, c, h, w = z.shape
    z_flat = jnp.transpose(z, (0, 2, 3, 1)).reshape(-1, c)
    cbt = codebook.astype(jnp.float32).T
    idx_flat, vq_loss = _dist_argmin(z_flat.astype(jnp.float32), cbt)
    z_q_flat = _sc_gather(codebook, idx_flat)
    z_q = jnp.transpose(z_q_flat.reshape(b, h, w, c), (0, 3, 1, 2))
    z_q_st = z + lax.stop_gradient(z_q - z)
    return (z_q_st, vq_loss, idx_flat.reshape(b, h, w))
